# Initial kernel scaffold; baseline (speedup 1.0000x reference)
import functools

import jax
import jax.numpy as jnp
from jax import lax
from jax.experimental import pallas as pl
from jax.experimental.pallas import tpu as pltpu
from jax.experimental.pallas import tpu_sc as plsc

_NUM_EMB = 1000000
_D = 32
_NW = 32
_L = 16


@functools.lru_cache(maxsize=None)
def _build(B):
    BPW = B // _NW
    C = 1024
    NCH = BPW // C
    assert BPW % C == 0 and BPW % _L == 0

    mesh = plsc.VectorSubcoreMesh(core_axis_name="c", subcore_axis_name="s")

    @functools.partial(
        pl.kernel,
        out_type=jax.ShapeDtypeStruct((B, _D), jnp.float32),
        mesh=mesh,
        scratch_types=[
            pltpu.VMEM((BPW,), jnp.int32),
            pltpu.VMEM((C, _D), jnp.float32),
            pltpu.SemaphoreType.DMA,
        ],
    )
    def k(x_hbm, tab_hbm, out_hbm, idx_v, rows, gsem):
        wid = lax.axis_index("s") * 2 + lax.axis_index("c")
        base = wid * BPW
        pltpu.sync_copy(x_hbm.at[pl.ds(base, BPW)], idx_v)

        def clamp_body(i, carry):
            sl = pl.ds(i * _L, _L)
            v = idx_v[sl]
            idx_v[sl] = jnp.minimum(jnp.maximum(v, 0), _NUM_EMB - 1)
            return carry

        lax.fori_loop(0, BPW // _L, clamp_body, 0)

        def chunk(c, carry):
            off = c * C
            pltpu.async_copy(
                tab_hbm.at[idx_v.at[pl.ds(off, C)]], rows, gsem
            ).wait()
            pltpu.sync_copy(rows, out_hbm.at[pl.ds(base + off, C)])
            return carry

        lax.fori_loop(0, NCH, chunk, 0)

    return k


def kernel(x, embedding_table):
    shape = x.shape
    xf = jnp.ravel(x).astype(jnp.int32)
    out = _build(xf.shape[0])(xf, embedding_table)
    return out.reshape(shape + (_D,))


# speedup vs baseline: 1.5521x; 1.5521x over previous
import functools

import jax
import jax.numpy as jnp
from jax import lax
from jax.experimental import pallas as pl
from jax.experimental.pallas import tpu as pltpu
from jax.experimental.pallas import tpu_sc as plsc

_NUM_EMB = 1000000
_D = 32
_NW = 32
_L = 16


@functools.lru_cache(maxsize=None)
def _build(B):
    BPW = B // _NW
    C = 1024
    NCH = BPW // C
    assert BPW % C == 0 and BPW % _L == 0

    mesh = plsc.VectorSubcoreMesh(core_axis_name="c", subcore_axis_name="s")

    @functools.partial(
        pl.kernel,
        out_type=jax.ShapeDtypeStruct((B, _D), jnp.float32),
        mesh=mesh,
        scratch_types=[
            pltpu.VMEM((BPW,), jnp.int32),
            pltpu.VMEM((C, _D), jnp.float32),
            pltpu.SemaphoreType.DMA,
        ],
        compiler_params=pltpu.CompilerParams(use_tc_tiling_on_sc=False),
    )
    def k(x_hbm, tab_hbm, out_hbm, idx_v, rows, gsem):
        wid = lax.axis_index("s") * 2 + lax.axis_index("c")
        base = wid * BPW
        pltpu.sync_copy(x_hbm.at[pl.ds(base, BPW)], idx_v)

        def clamp_body(i, carry):
            sl = pl.ds(i * _L, _L)
            v = idx_v[sl]
            idx_v[sl] = jnp.minimum(jnp.maximum(v, 0), _NUM_EMB - 1)
            return carry

        lax.fori_loop(0, BPW // _L, clamp_body, 0)

        def chunk(c, carry):
            off = c * C
            pltpu.async_copy(
                tab_hbm.at[idx_v.at[pl.ds(off, C)]], rows, gsem
            ).wait()
            pltpu.sync_copy(rows, out_hbm.at[pl.ds(base + off, C)])
            return carry

        lax.fori_loop(0, NCH, chunk, 0)

    return k


def kernel(x, embedding_table):
    shape = x.shape
    xf = jnp.ravel(x).astype(jnp.int32)
    out = _build(xf.shape[0])(xf, embedding_table)
    return out.reshape(shape + (_D,))


# speedup vs baseline: 1.5776x; 1.0164x over previous
import functools

import jax
import jax.numpy as jnp
from jax import lax
from jax.experimental import pallas as pl
from jax.experimental.pallas import tpu as pltpu
from jax.experimental.pallas import tpu_sc as plsc

_NUM_EMB = 1000000
_D = 32
_NW = 32
_L = 16
_NBUF = 6
_DEPTH = 4
_C = 512


@functools.lru_cache(maxsize=None)
def _build(B):
    BPW = B // _NW
    NCH = BPW // _C
    assert BPW % _C == 0 and NCH > _NBUF

    mesh = plsc.VectorSubcoreMesh(core_axis_name="c", subcore_axis_name="s")

    @functools.partial(
        pl.kernel,
        out_type=jax.ShapeDtypeStruct((B, _D), jnp.float32),
        mesh=mesh,
        scratch_types=[
            pltpu.VMEM((BPW,), jnp.int32),
            *[pltpu.VMEM((_C, _D), jnp.float32) for _ in range(_NBUF)],
            *[pltpu.SemaphoreType.DMA for _ in range(2 * _NBUF)],
        ],
        compiler_params=pltpu.CompilerParams(use_tc_tiling_on_sc=False),
    )
    def k(x_hbm, tab_hbm, out_hbm, idx_v, *rest):
        bufs = rest[:_NBUF]
        gsems = rest[_NBUF:2 * _NBUF]
        wsems = rest[2 * _NBUF:]

        wid = lax.axis_index("s") * 2 + lax.axis_index("c")
        base = wid * BPW
        pltpu.sync_copy(x_hbm.at[pl.ds(base, BPW)], idx_v)

        def clamp_chunk(c):
            def body(i, carry):
                sl = pl.ds(c * _C + i * _L, _L)
                v = idx_v[sl]
                idx_v[sl] = jnp.minimum(jnp.maximum(v, 0), _NUM_EMB - 1)
                return carry

            lax.fori_loop(0, _C // _L, body, 0)

        gd = [None] * _NBUF
        wd = [None] * _NBUF

        def fire_gather(c):
            b = c % _NBUF
            clamp_chunk(c)
            gd[b] = pltpu.async_copy(
                tab_hbm.at[idx_v.at[pl.ds(c * _C, _C)]], bufs[b], gsems[b]
            )

        for j in range(_DEPTH):
            fire_gather(j)
        for c in range(NCH):
            b = c % _NBUF
            if c + _DEPTH < NCH:
                pb = (c + _DEPTH) % _NBUF
                if c + _DEPTH - _NBUF >= 0:
                    wd[pb].wait()
                fire_gather(c + _DEPTH)
            gd[b].wait()
            wd[b] = pltpu.async_copy(
                bufs[b], out_hbm.at[pl.ds(base + c * _C, _C)], wsems[b]
            )
        for b in range(_NBUF):
            wd[b].wait()

    return k


def kernel(x, embedding_table):
    shape = x.shape
    xf = jnp.ravel(x).astype(jnp.int32)
    out = _build(xf.shape[0])(xf, embedding_table)
    return out.reshape(shape + (_D,))


# speedup vs baseline: 1.5814x; 1.0024x over previous
import functools

import jax
import jax.numpy as jnp
from jax import lax
from jax.experimental import pallas as pl
from jax.experimental.pallas import tpu as pltpu
from jax.experimental.pallas import tpu_sc as plsc

_NUM_EMB = 1000000
_D = 32
_NW = 32
_L = 16
_NBUF = 6
_DEPTH = 4


@functools.lru_cache(maxsize=None)
def _build(T, J):
    B = T * J
    TPW = T // _NW
    BPW = B // _NW
    NCH = J
    assert TPW % _L == 0 and TPW % 8 == 0 and NCH > _NBUF

    mesh = plsc.VectorSubcoreMesh(core_axis_name="c", subcore_axis_name="s")

    @functools.partial(
        pl.kernel,
        out_type=jax.ShapeDtypeStruct((T, J, _D), jnp.float32),
        mesh=mesh,
        scratch_types=[
            pltpu.VMEM((BPW,), jnp.int32),
            *[pltpu.VMEM((TPW, _D), jnp.float32) for _ in range(_NBUF)],
            *[pltpu.SemaphoreType.DMA for _ in range(2 * _NBUF)],
        ],
        compiler_params=pltpu.CompilerParams(use_tc_tiling_on_sc=False),
    )
    def k(x_hbm, tab_hbm, out_hbm, idx_v, *rest):
        bufs = rest[:_NBUF]
        gsems = rest[_NBUF:2 * _NBUF]
        wsems = rest[2 * _NBUF:]

        wid = lax.axis_index("s") * 2 + lax.axis_index("c")
        base = wid * BPW
        t_base = wid * TPW
        pltpu.sync_copy(x_hbm.at[pl.ds(base, BPW)], idx_v)

        def clamp_chunk(c):
            def body(i, carry):
                sl = pl.ds(c * TPW + i * _L, _L)
                v = idx_v[sl]
                idx_v[sl] = jnp.minimum(jnp.maximum(v, 0), _NUM_EMB - 1)
                return carry

            lax.fori_loop(0, TPW // _L, body, 0)

        gd = [None] * _NBUF
        wd = [None] * _NBUF

        def fire_gather(c):
            b = c % _NBUF
            clamp_chunk(c)
            gd[b] = pltpu.async_copy(
                tab_hbm.at[idx_v.at[pl.ds(c * TPW, TPW)]], bufs[b], gsems[b]
            )

        for j in range(_DEPTH):
            fire_gather(j)
        for c in range(NCH):
            b = c % _NBUF
            if c + _DEPTH < NCH:
                pb = (c + _DEPTH) % _NBUF
                if c + _DEPTH - _NBUF >= 0:
                    wd[pb].wait()
                fire_gather(c + _DEPTH)
            gd[b].wait()
            wd[b] = pltpu.async_copy(
                bufs[b], out_hbm.at[pl.ds(t_base, TPW), c], wsems[b]
            )
        for b in range(_NBUF):
            wd[b].wait()

    return k


def kernel(x, embedding_table):
    T, J = x.shape
    xp = jnp.transpose(
        jnp.reshape(x.astype(jnp.int32), (_NW, T // _NW, J)), (0, 2, 1)
    ).reshape(-1)
    return _build(T, J)(xp, embedding_table)
